# baseline (speedup 1.0000x reference)
import math
import functools

import jax
import jax.numpy as jnp
from jax import lax
from jax.experimental import pallas as pl
from jax.experimental.pallas import tpu as pltpu
from jax.experimental.pallas import tpu_sc as plsc

IN_DIM = 258
OUT_DIM = 2048
SEQ_LEN = 1000

BM = 512
CH = 16


def _colstats_body(x_ref, o_ref, *, vmax):
    col = jnp.clip(x_ref[:, 1:2].astype(jnp.int32), 0, vmax)
    lanes = lax.broadcasted_iota(jnp.int32, (1, 1, 128), 2)
    o_ref[...] = jnp.where(lanes < 64, jnp.min(col), jnp.max(col))


def _colstats(x_flat, vmax):
    m = x_flat.shape[0]
    nb = m // BM
    return pl.pallas_call(
        functools.partial(_colstats_body, vmax=vmax),
        grid=(nb,),
        in_specs=[pl.BlockSpec((BM, 128), lambda i: (i, 0))],
        out_specs=pl.BlockSpec((1, 1, 128), lambda i: (i, 0, 0)),
        out_shape=jax.ShapeDtypeStruct((nb, 1, 128), jnp.int32),
    )(x_flat)


@functools.partial(jax.jit, static_argnums=(3,))
def _sc_lookup(emb, stats, x_flat, m):
    nb = m // BM
    info = plsc.get_sparse_core_info()
    nw = info.num_cores * info.num_subcores
    bpw = nb // nw
    vmax = emb.shape[0] - 1
    n_chunks = BM // CH
    mesh = plsc.VectorSubcoreMesh(
        core_axis_name="c", subcore_axis_name="s",
        num_cores=info.num_cores, num_subcores=info.num_subcores
    )

    @functools.partial(
        pl.kernel,
        mesh=mesh,
        out_type=[
            jax.ShapeDtypeStruct((nb, 16), jnp.int32),
            jax.ShapeDtypeStruct((nb, 1, OUT_DIM), jnp.float32),
            jax.ShapeDtypeStruct((m, OUT_DIM), jnp.float32),
        ],
        compiler_params=pltpu.CompilerParams(needs_layout_passes=False),
        scratch_types=[
            pltpu.VMEM((BM,), jnp.int32),
            pltpu.VMEM((1, 128), jnp.int32),
            pltpu.VMEM((16, 128), jnp.float32),
            pltpu.VMEM((1, 16), jnp.int32),
            pltpu.VMEM((1, OUT_DIM), jnp.float32),
            pltpu.VMEM((2, CH, OUT_DIM), jnp.float32),
            pltpu.SemaphoreType.DMA,
            pltpu.SemaphoreType.DMA,
        ],
    )
    def k(emb_hbm, stats_hbm, x_hbm, flags_hbm, brow_hbm, ovf_hbm,
          idx_v, stats_v, xchunk_v, flag_v, row_v, rows_v, sem_g, sem_s):
        wid = lax.axis_index("s") * info.num_cores + lax.axis_index("c")

        for j in range(bpw):
            blk = wid * bpw + j
            base = blk * BM
            pltpu.sync_copy(stats_hbm.at[blk], stats_v)
            mn_s = stats_v[0, pl.ds(0, 16)][0]
            mx_s = stats_v[0, pl.ds(64, 16)][0]
            uniform = (mn_s == mx_s).astype(jnp.int32)

            flag_v[0, :] = jnp.zeros((16,), jnp.int32) + uniform
            pltpu.sync_copy(flag_v, flags_hbm.at[pl.ds(blk, 1)])

            @pl.when(uniform == 1)
            def _():
                pltpu.sync_copy(emb_hbm.at[pl.ds(mn_s, 1)], row_v)
                pltpu.sync_copy(row_v, brow_hbm.at[blk])

            @pl.when(uniform == 0)
            def _():
                rows16 = lax.iota(jnp.int32, 16)
                col1 = jnp.ones((16,), jnp.int32)

                def ext_body(c, carry):
                    pltpu.sync_copy(
                        x_hbm.at[pl.ds(base + c * 16, 16), pl.ds(0, 128)],
                        xchunk_v,
                    )
                    vals = plsc.load_gather(xchunk_v, [rows16, col1])
                    idx_v[pl.ds(c * 16, 16)] = jnp.clip(
                        vals.astype(jnp.int32), 0, vmax
                    )
                    return carry

                lax.fori_loop(0, BM // 16, ext_body, 0)

                def gather(c):
                    return pltpu.make_async_copy(
                        emb_hbm.at[idx_v.at[pl.ds(c * CH, CH)]],
                        rows_v.at[c % 2],
                        sem_g,
                    )

                def scatter(c):
                    return pltpu.make_async_copy(
                        rows_v.at[c % 2],
                        ovf_hbm.at[pl.ds(base + c * CH, CH)],
                        sem_s,
                    )

                gather(0).start()

                def body(c, carry):
                    @pl.when(c >= 1)
                    def _():
                        scatter(c - 1).wait()

                    @pl.when(c + 1 < n_chunks)
                    def _():
                        gather(c + 1).start()

                    gather(c).wait()
                    scatter(c).start()
                    return carry

                lax.fori_loop(0, n_chunks, body, 0)
                scatter(n_chunks - 1).wait()

    return k(emb, stats, x_flat)


def _mm_body(flags_ref, x_ref, wt_ref, b_ref, brow_ref, ovf_ref, o_ref,
             ovf_v, sem):
    i = pl.program_id(0)
    flag = flags_ref[i, 0]

    @pl.when(flag == 0)
    def _():
        pltpu.make_async_copy(
            ovf_ref.at[pl.ds(i * BM, BM)], ovf_v, sem
        ).start()

    xb = x_ref[...].astype(jnp.bfloat16)
    acc = jnp.dot(xb, wt_ref[...], preferred_element_type=jnp.float32)
    acc = acc + b_ref[...]

    @pl.when(flag == 1)
    def _():
        o_ref[...] = acc + brow_ref[0]

    @pl.when(flag == 0)
    def _():
        pltpu.make_async_copy(
            ovf_ref.at[pl.ds(i * BM, BM)], ovf_v, sem
        ).wait()
        o_ref[...] = acc + ovf_v[...]


def _matmul_add(x_flat, wt, b_aug, flags, block_rows, overflow):
    m = x_flat.shape[0]
    grid = (m // BM,)
    return pl.pallas_call(
        _mm_body,
        grid_spec=pltpu.PrefetchScalarGridSpec(
            num_scalar_prefetch=1,
            grid=grid,
            in_specs=[
                pl.BlockSpec((BM, IN_DIM), lambda i, f: (i, 0)),
                pl.BlockSpec((IN_DIM, OUT_DIM), lambda i, f: (0, 0)),
                pl.BlockSpec((1, OUT_DIM), lambda i, f: (0, 0)),
                pl.BlockSpec((1, 1, OUT_DIM), lambda i, f: (i, 0, 0)),
                pl.BlockSpec(memory_space=pl.ANY),
            ],
            out_specs=pl.BlockSpec((BM, OUT_DIM), lambda i, f: (i, 0)),
            scratch_shapes=[
                pltpu.VMEM((BM, OUT_DIM), jnp.float32),
                pltpu.SemaphoreType.DMA,
            ],
        ),
        out_shape=jax.ShapeDtypeStruct((m, OUT_DIM), jnp.float32),
    )(flags, x_flat, wt, b_aug, block_rows, overflow)


def kernel(x, W_epoch, emb, W_conf, b_conf):
    B, S, _ = x.shape
    m = B * S
    x_flat = x.reshape(m, IN_DIM)

    scale = math.sqrt(12.0) / float(SEQ_LEN)
    w_ep = W_epoch[:, 0]
    wt = jnp.concatenate(
        [
            (w_ep * scale)[None, :],
            jnp.zeros((1, OUT_DIM), jnp.float32),
            W_conf.T,
        ],
        axis=0,
    ).astype(jnp.bfloat16)
    b_aug = (b_conf - 0.5 * math.sqrt(12.0) * w_ep)[None, :]

    stats = _colstats(x_flat, emb.shape[0] - 1)
    flags, block_rows, overflow = _sc_lookup(emb, stats, x_flat, m)

    out = _matmul_add(x_flat, wt, b_aug, flags, block_rows, overflow)
    return out.reshape(B, S, OUT_DIM)


# speedup vs baseline: 1.2857x; 1.2857x over previous
import math
import functools

import jax
import jax.numpy as jnp
from jax import lax
from jax.experimental import pallas as pl
from jax.experimental.pallas import tpu as pltpu
from jax.experimental.pallas import tpu_sc as plsc

IN_DIM = 258
OUT_DIM = 2048
SEQ_LEN = 1000

BM = 1024
CH = 16


@functools.partial(jax.jit, static_argnums=(2,))
def _sc_lookup(emb, idx, m):
    nb = m // BM
    info = plsc.get_sparse_core_info()
    nw = info.num_cores * info.num_subcores
    bpw = nb // nw
    vmax = emb.shape[0] - 1
    n_chunks = BM // CH
    mesh = plsc.VectorSubcoreMesh(
        core_axis_name="c", subcore_axis_name="s",
        num_cores=info.num_cores, num_subcores=info.num_subcores
    )

    @functools.partial(
        pl.kernel,
        mesh=mesh,
        out_type=[
            jax.ShapeDtypeStruct((nb, 16), jnp.int32),
            jax.ShapeDtypeStruct((nb, 1, OUT_DIM), jnp.float32),
            jax.ShapeDtypeStruct((m, OUT_DIM), jnp.float32),
        ],
        compiler_params=pltpu.CompilerParams(needs_layout_passes=False),
        scratch_types=[
            pltpu.VMEM((BM,), jnp.int32),
            pltpu.VMEM((16,), jnp.int32),
            pltpu.VMEM((1, 16), jnp.int32),
            pltpu.VMEM((1, OUT_DIM), jnp.float32),
            pltpu.VMEM((2, CH, OUT_DIM), jnp.float32),
            pltpu.SemaphoreType.DMA,
            pltpu.SemaphoreType.DMA,
        ],
    )
    def k(emb_hbm, idx_hbm, flags_hbm, brow_hbm, ovf_hbm,
          idx_v, red_v, flag_v, row_v, rows_v, sem_g, sem_s):
        wid = lax.axis_index("s") * info.num_cores + lax.axis_index("c")

        for j in range(bpw):
            blk = wid * bpw + j
            base = blk * BM
            pltpu.sync_copy(idx_hbm.at[pl.ds(base, BM)], idx_v)

            def mm_body(c, carry):
                mn, mx = carry
                v = idx_v[pl.ds(c * 16, 16)]
                v = jnp.clip(v, 0, vmax)
                idx_v[pl.ds(c * 16, 16)] = v
                return jnp.minimum(mn, v), jnp.maximum(mx, v)

            init = (jnp.full((16,), vmax, jnp.int32),
                    jnp.zeros((16,), jnp.int32))
            mn, mx = lax.fori_loop(0, BM // 16, mm_body, init)

            lanes = lax.iota(jnp.int32, 16)
            for kk in (1, 2, 4, 8):
                rot = jnp.bitwise_and(lanes + kk, 15)
                red_v[...] = mn
                mn = jnp.minimum(mn, plsc.load_gather(red_v, [rot]))
                red_v[...] = mx
                mx = jnp.maximum(mx, plsc.load_gather(red_v, [rot]))
            mn_s = mn[0]
            mx_s = mx[0]
            uniform = (mn_s == mx_s).astype(jnp.int32)

            flag_v[0, :] = (mn == mx).astype(jnp.int32)
            pltpu.sync_copy(flag_v, flags_hbm.at[pl.ds(blk, 1)])

            @pl.when(uniform == 1)
            def _():
                pltpu.sync_copy(emb_hbm.at[pl.ds(mn_s, 1)], row_v)
                pltpu.sync_copy(row_v, brow_hbm.at[blk])

            @pl.when(uniform == 0)
            def _():
                def gather(c):
                    return pltpu.make_async_copy(
                        emb_hbm.at[idx_v.at[pl.ds(c * CH, CH)]],
                        rows_v.at[c % 2],
                        sem_g,
                    )

                def scatter(c):
                    return pltpu.make_async_copy(
                        rows_v.at[c % 2],
                        ovf_hbm.at[pl.ds(base + c * CH, CH)],
                        sem_s,
                    )

                gather(0).start()

                def body(c, carry):
                    @pl.when(c >= 1)
                    def _():
                        scatter(c - 1).wait()

                    @pl.when(c + 1 < n_chunks)
                    def _():
                        gather(c + 1).start()

                    gather(c).wait()
                    scatter(c).start()
                    return carry

                lax.fori_loop(0, n_chunks, body, 0)
                scatter(n_chunks - 1).wait()

    return k(emb, idx)


def _mm_body(flags_ref, x_ref, wt_ref, b_ref, brow_ref, ovf_ref, o_ref,
             ovf_v, sem):
    i = pl.program_id(0)
    flag = flags_ref[i, 0]

    @pl.when(flag == 0)
    def _():
        pltpu.make_async_copy(
            ovf_ref.at[pl.ds(i * BM, BM)], ovf_v, sem
        ).start()

    xb = x_ref[...].astype(jnp.bfloat16)
    acc = jnp.dot(xb, wt_ref[...], preferred_element_type=jnp.float32)
    acc = acc + b_ref[...]

    @pl.when(flag == 1)
    def _():
        o_ref[...] = acc + brow_ref[0]

    @pl.when(flag == 0)
    def _():
        pltpu.make_async_copy(
            ovf_ref.at[pl.ds(i * BM, BM)], ovf_v, sem
        ).wait()
        o_ref[...] = acc + ovf_v[...]


def _matmul_add(x_flat, wt, b_aug, flags, block_rows, overflow):
    m = x_flat.shape[0]
    grid = (m // BM,)
    return pl.pallas_call(
        _mm_body,
        grid_spec=pltpu.PrefetchScalarGridSpec(
            num_scalar_prefetch=1,
            grid=grid,
            in_specs=[
                pl.BlockSpec((BM, IN_DIM), lambda i, f: (i, 0)),
                pl.BlockSpec((IN_DIM, OUT_DIM), lambda i, f: (0, 0)),
                pl.BlockSpec((1, OUT_DIM), lambda i, f: (0, 0)),
                pl.BlockSpec((1, 1, OUT_DIM), lambda i, f: (i, 0, 0)),
                pl.BlockSpec(memory_space=pl.ANY),
            ],
            out_specs=pl.BlockSpec((BM, OUT_DIM), lambda i, f: (i, 0)),
            scratch_shapes=[
                pltpu.VMEM((BM, OUT_DIM), jnp.float32),
                pltpu.SemaphoreType.DMA,
            ],
        ),
        out_shape=jax.ShapeDtypeStruct((m, OUT_DIM), jnp.float32),
    )(flags, x_flat, wt, b_aug, block_rows, overflow)


def kernel(x, W_epoch, emb, W_conf, b_conf):
    B, S, _ = x.shape
    m = B * S
    x_flat = x.reshape(m, IN_DIM)

    scale = math.sqrt(12.0) / float(SEQ_LEN)
    w_ep = W_epoch[:, 0]
    wt = jnp.concatenate(
        [
            (w_ep * scale)[None, :],
            jnp.zeros((1, OUT_DIM), jnp.float32),
            W_conf.T,
        ],
        axis=0,
    ).astype(jnp.bfloat16)
    b_aug = (b_conf - 0.5 * math.sqrt(12.0) * w_ep)[None, :]

    idx = x_flat[:, 1].astype(jnp.int32)
    flags, block_rows, overflow = _sc_lookup(emb, idx, m)

    out = _matmul_add(x_flat, wt, b_aug, flags, block_rows, overflow)
    return out.reshape(B, S, OUT_DIM)
